# baseline (speedup 1.0000x reference)
import jax
import jax.numpy as jnp
from jax.experimental import pallas as pl
from jax.experimental.pallas import tpu as pltpu

_EPS = 1e-8
_STREAMS = 8
_FTILE = 512


def _norm_protos(ca):
    ca_sq = jnp.sum(ca * ca, axis=1, keepdims=True)
    ca_inv = 1.0 / jnp.maximum(jnp.sqrt(ca_sq), _EPS)
    return ca * ca_inv


def _cos_tile(x, ca_unit):
    raw = jax.lax.dot_general(
        x, ca_unit, (((1,), (1,)), ((), ())),
        preferred_element_type=jnp.float32)
    row_sq = jnp.sum(x * x, axis=1, keepdims=True)
    inv = 1.0 / jnp.maximum(jnp.sqrt(row_sq), _EPS)
    return raw * inv


def _fast_tile(flag_ref, *refs):
    lab_ref = refs[_STREAMS]
    ca_ref = refs[_STREAMS + 1]
    o_ref = refs[_STREAMS + 2]
    ca = ca_ref[...]
    k = ca.shape[0]
    ca_unit = _norm_protos(ca)
    for s in range(_STREAMS):
        o_ref[s] = _cos_tile(refs[s][0], ca_unit)

    @pl.when(flag_ref[0] != 0)
    def _override():
        inf_mask = jnp.where(jnp.isinf(ca), 1.0, 0.0)
        ones_row = jnp.ones((1, ca.shape[1]), jnp.float32)
        has_inf = jax.lax.dot_general(
            ones_row, inf_mask, (((1,), (1,)), ((), ())),
            preferred_element_type=jnp.float32) > 0.0
        labl = lab_ref[0].astype(jnp.float32)
        ti = jax.lax.broadcasted_iota(jnp.int32, (_FTILE, 128), 0)
        ci = jax.lax.broadcasted_iota(jnp.int32, (_FTILE, 128), 1)
        sel = jnp.where(ti // 4 == ci, 1.0, 0.0)
        call = jax.lax.dot_general(
            sel, labl, (((1,), (0,)), ((), ())),
            preferred_element_type=jnp.float32)
        tmod = jax.lax.broadcasted_iota(jnp.int32, (_FTILE, call.shape[1]), 0) % 4
        lane = jax.lax.broadcasted_iota(jnp.int32, (_FTILE, call.shape[1]), 1)
        kidx = jax.lax.broadcasted_iota(jnp.int32, (_FTILE, k), 1)
        for s in range(_STREAMS):
            pick = jnp.where(lane == 4 * s + tmod, 1.0, 0.0)
            labs = jnp.sum(call * pick, axis=1, keepdims=True)
            uninit = jnp.where(labs.astype(jnp.int32) == kidx, 1.0, -1.0)
            o_ref[s] = jnp.where(has_inf, uninit, o_ref[s])


def kernel(inputs, labels, class_avgs):
    b, t, d = inputs.shape
    k = class_avgs.shape[0]
    rows = (b * t) // _STREAMS
    x3 = inputs.reshape(_STREAMS, rows, d)
    nsteps = rows // _FTILE
    labs3 = (labels.astype(jnp.int32)
             .reshape(_STREAMS, nsteps, 128, 4)
             .transpose(1, 2, 0, 3)
             .reshape(nsteps, 128, _STREAMS * 4))
    flag = jnp.any(jnp.isinf(class_avgs)).astype(jnp.int32).reshape(1)

    grid = (rows // _FTILE,)
    xspec = lambda si: pl.BlockSpec((1, _FTILE, d), lambda i, *_: (si, i, 0))
    grid_spec = pltpu.PrefetchScalarGridSpec(
        num_scalar_prefetch=1,
        grid=grid,
        in_specs=[xspec(s) for s in range(_STREAMS)] + [
            pl.BlockSpec((1, 128, _STREAMS * 4),
                         lambda i, *_: (i, 0, 0)),
            pl.BlockSpec((k, d), lambda i, *_: (0, 0)),
        ],
        out_specs=pl.BlockSpec((_STREAMS, _FTILE, k),
                               lambda i, *_: (0, i, 0)),
    )
    out = pl.pallas_call(
        _fast_tile,
        grid_spec=grid_spec,
        out_shape=jax.ShapeDtypeStruct((_STREAMS, rows, k), jnp.float32),
        compiler_params=pltpu.CompilerParams(
            dimension_semantics=("arbitrary",)),
    )(flag, *([x3] * _STREAMS), labs3, class_avgs)
    return out.reshape(b, t, k)


# speedup vs baseline: 1.0233x; 1.0233x over previous
import jax
import jax.numpy as jnp
from jax.experimental import pallas as pl
from jax.experimental.pallas import tpu as pltpu

_EPS = 1e-8
_STREAMS = 8
_FTILE = 512


def _norm_protos(ca):
    ca_sq = jnp.sum(ca * ca, axis=1, keepdims=True)
    ca_inv = 1.0 / jnp.maximum(jnp.sqrt(ca_sq), _EPS)
    return ca * ca_inv


def _cos_tile(x, ca_unit):
    raw = jax.lax.dot_general(
        x, ca_unit, (((1,), (1,)), ((), ())),
        preferred_element_type=jnp.float32)
    row_sq = jnp.sum(x * x, axis=1, keepdims=True)
    inv = 1.0 / jnp.maximum(jnp.sqrt(row_sq), _EPS)
    return raw * inv


def _fast_tile(*refs):
    lab_ref = refs[_STREAMS]
    ca_ref = refs[_STREAMS + 1]
    o_ref = refs[_STREAMS + 2]
    ca = ca_ref[...]
    k = ca.shape[0]

    inf_mask = jnp.where(jnp.isinf(ca), 1.0, 0.0)
    ones_row = jnp.ones((1, ca.shape[1]), jnp.float32)
    has_inf = jax.lax.dot_general(
        ones_row, inf_mask, (((1,), (1,)), ((), ())),
        preferred_element_type=jnp.float32) > 0.0
    col_has_inf = jnp.max(inf_mask, axis=1, keepdims=True) > 0.0
    ca_unit = _norm_protos(jnp.where(col_has_inf, 0.0, ca))

    labl = lab_ref[0].astype(jnp.float32)
    ti = jax.lax.broadcasted_iota(jnp.int32, (_FTILE, 128), 0)
    ci = jax.lax.broadcasted_iota(jnp.int32, (_FTILE, 128), 1)
    sel = jnp.where(ti // 4 == ci, 1.0, 0.0)
    call = jax.lax.dot_general(
        sel, labl, (((1,), (0,)), ((), ())),
        preferred_element_type=jnp.float32)
    nl = call.shape[1]
    tmod = jax.lax.broadcasted_iota(jnp.int32, (_FTILE, nl), 0) % 4
    lane = jax.lax.broadcasted_iota(jnp.int32, (_FTILE, nl), 1)
    kidx = jax.lax.broadcasted_iota(jnp.int32, (_FTILE, k), 1)

    for s in range(_STREAMS):
        cos = _cos_tile(refs[s][0], ca_unit)
        pick = jnp.where(lane == 4 * s + tmod, 1.0, 0.0)
        labs = jnp.sum(call * pick, axis=1, keepdims=True)
        uninit = jnp.where(labs.astype(jnp.int32) == kidx, 1.0, -1.0)
        o_ref[s] = jnp.where(has_inf, uninit, cos)


def kernel(inputs, labels, class_avgs):
    b, t, d = inputs.shape
    k = class_avgs.shape[0]
    rows = (b * t) // _STREAMS
    x3 = inputs.reshape(_STREAMS, rows, d)
    nsteps = rows // _FTILE
    labs3 = (labels.astype(jnp.int32)
             .reshape(_STREAMS, nsteps, 128, 4)
             .transpose(1, 2, 0, 3)
             .reshape(nsteps, 128, _STREAMS * 4))

    grid = (nsteps,)
    xspec = lambda si: pl.BlockSpec((1, _FTILE, d), lambda i: (si, i, 0))
    out = pl.pallas_call(
        _fast_tile,
        grid=grid,
        in_specs=[xspec(s) for s in range(_STREAMS)] + [
            pl.BlockSpec((1, 128, _STREAMS * 4), lambda i: (i, 0, 0)),
            pl.BlockSpec((k, d), lambda i: (0, 0)),
        ],
        out_specs=pl.BlockSpec((_STREAMS, _FTILE, k), lambda i: (0, i, 0)),
        out_shape=jax.ShapeDtypeStruct((_STREAMS, rows, k), jnp.float32),
        compiler_params=pltpu.CompilerParams(
            dimension_semantics=("arbitrary",)),
    )(*([x3] * _STREAMS), labs3, class_avgs)
    return out.reshape(b, t, k)


# speedup vs baseline: 1.1194x; 1.0940x over previous
import jax
import jax.numpy as jnp
from jax.experimental import pallas as pl
from jax.experimental.pallas import tpu as pltpu

_EPS = 1e-8
_STREAMS = 8
_FTILE = 512


def _norm_protos(ca):
    ca_sq = jnp.sum(ca * ca, axis=1, keepdims=True)
    ca_inv = 1.0 / jnp.maximum(jnp.sqrt(ca_sq), _EPS)
    return ca * ca_inv


def _cos_tile(x, ca_unit):
    raw = jax.lax.dot_general(
        x, ca_unit, (((1,), (1,)), ((), ())),
        preferred_element_type=jnp.float32)
    row_sq = jnp.sum(x * x, axis=1, keepdims=True)
    inv = 1.0 / jnp.maximum(jnp.sqrt(row_sq), _EPS)
    return raw * inv


def _fast_tile(*refs):
    lab_ref = refs[_STREAMS]
    ca_ref = refs[_STREAMS + 1]
    o_ref = refs[_STREAMS + 2]
    ca = ca_ref[...]
    k = ca.shape[0]

    inf_mask = jnp.where(jnp.isinf(ca), 1.0, 0.0)
    ones_row = jnp.ones((1, ca.shape[1]), jnp.float32)
    has_inf = jax.lax.dot_general(
        ones_row, inf_mask, (((1,), (1,)), ((), ())),
        preferred_element_type=jnp.float32) > 0.0
    col_has_inf = jnp.max(inf_mask, axis=1, keepdims=True) > 0.0
    ca_unit = _norm_protos(jnp.where(col_has_inf, 0.0, ca))

    ti = jax.lax.broadcasted_iota(jnp.int32, (_FTILE, 4), 0)
    ri = jax.lax.broadcasted_iota(jnp.int32, (_FTILE, 4), 1)
    sel = jnp.where(ti // 128 == ri, 1.0, 0.0)
    tmod = jax.lax.broadcasted_iota(jnp.int32, (_FTILE, 128), 0) % 128
    lane = jax.lax.broadcasted_iota(jnp.int32, (_FTILE, 128), 1)
    pick = jnp.where(lane == tmod, 1.0, 0.0)
    kidx = jax.lax.broadcasted_iota(jnp.int32, (_FTILE, k), 1)

    for s in range(_STREAMS):
        cos = _cos_tile(refs[s][0], ca_unit)
        labl = lab_ref[s, 0].astype(jnp.float32)
        call = jax.lax.dot_general(
            sel, labl, (((1,), (0,)), ((), ())),
            preferred_element_type=jnp.float32)
        labs = jnp.sum(call * pick, axis=1, keepdims=True)
        uninit = jnp.where(labs.astype(jnp.int32) == kidx, 1.0, -1.0)
        o_ref[s] = jnp.where(has_inf, uninit, cos)


def kernel(inputs, labels, class_avgs):
    b, t, d = inputs.shape
    k = class_avgs.shape[0]
    rows = (b * t) // _STREAMS
    x3 = inputs.reshape(_STREAMS, rows, d)
    nsteps = rows // _FTILE
    labs4 = labels.astype(jnp.int32).reshape(_STREAMS, nsteps, 4, 128)

    grid = (nsteps,)
    xspec = lambda si: pl.BlockSpec((1, _FTILE, d), lambda i: (si, i, 0))
    out = pl.pallas_call(
        _fast_tile,
        grid=grid,
        in_specs=[xspec(s) for s in range(_STREAMS)] + [
            pl.BlockSpec((_STREAMS, 1, 4, 128), lambda i: (0, i, 0, 0)),
            pl.BlockSpec((k, d), lambda i: (0, 0)),
        ],
        out_specs=pl.BlockSpec((_STREAMS, _FTILE, k), lambda i: (0, i, 0)),
        out_shape=jax.ShapeDtypeStruct((_STREAMS, rows, k), jnp.float32),
        compiler_params=pltpu.CompilerParams(
            dimension_semantics=("arbitrary",)),
    )(*([x3] * _STREAMS), labs4, class_avgs)
    return out.reshape(b, t, k)


# speedup vs baseline: 1.1891x; 1.0622x over previous
import jax
import jax.numpy as jnp
from jax.experimental import pallas as pl
from jax.experimental.pallas import tpu as pltpu

_EPS = 1e-8
_STREAMS = 8
_FTILE = 256
_NBUF = 3
_LROWS = _FTILE // 128


def _norm_protos(ca):
    ca_sq = jnp.sum(ca * ca, axis=1, keepdims=True)
    ca_inv = 1.0 / jnp.maximum(jnp.sqrt(ca_sq), _EPS)
    return ca * ca_inv


def _cos_tile(x, ca_unit):
    raw = jax.lax.dot_general(
        x, ca_unit, (((1,), (1,)), ((), ())),
        preferred_element_type=jnp.float32)
    row_sq = jnp.sum(x * x, axis=1, keepdims=True)
    inv = 1.0 / jnp.maximum(jnp.sqrt(row_sq), _EPS)
    return raw * inv


def _pipeline_body(x_hbm, lab_ref, ca_ref, o_hbm,
                   xbuf, obuf, in_sems, out_sems):
    nsteps = x_hbm.shape[1] // _FTILE
    ca = ca_ref[...]
    k = ca.shape[0]

    inf_mask = jnp.where(jnp.isinf(ca), 1.0, 0.0)
    ones_row = jnp.ones((1, ca.shape[1]), jnp.float32)
    has_inf = jax.lax.dot_general(
        ones_row, inf_mask, (((1,), (1,)), ((), ())),
        preferred_element_type=jnp.float32) > 0.0
    col_has_inf = jnp.max(inf_mask, axis=1, keepdims=True) > 0.0
    ca_unit = _norm_protos(jnp.where(col_has_inf, 0.0, ca))

    ti = jax.lax.broadcasted_iota(jnp.int32, (_FTILE, _LROWS), 0)
    ri = jax.lax.broadcasted_iota(jnp.int32, (_FTILE, _LROWS), 1)
    sel = jnp.where(ti // 128 == ri, 1.0, 0.0)
    tmod = jax.lax.broadcasted_iota(jnp.int32, (_FTILE, 128), 0) % 128
    lane = jax.lax.broadcasted_iota(jnp.int32, (_FTILE, 128), 1)
    pick = jnp.where(lane == tmod, 1.0, 0.0)
    kidx = jax.lax.broadcasted_iota(jnp.int32, (_FTILE, k), 1)

    def in_copies(j):
        slot = j % _NBUF
        return [pltpu.make_async_copy(
            x_hbm.at[s, pl.ds(j * _FTILE, _FTILE), :],
            xbuf.at[slot, s],
            in_sems.at[slot, s]) for s in range(_STREAMS)]

    def out_copy(j):
        return pltpu.make_async_copy(
            obuf.at[j % 2],
            o_hbm.at[:, pl.ds(j * _FTILE, _FTILE), :],
            out_sems.at[j % 2])

    for c in in_copies(0):
        c.start()
    for c in in_copies(1):
        c.start()

    for j in range(nsteps):
        if j + 2 < nsteps:
            for c in in_copies(j + 2):
                c.start()
        for c in in_copies(j):
            c.wait()
        if j >= 2:
            out_copy(j - 2).wait()
        slot = j % _NBUF
        for s in range(_STREAMS):
            cos = _cos_tile(xbuf[slot, s], ca_unit)
            labl = lab_ref[s, j].astype(jnp.float32)
            call = jax.lax.dot_general(
                sel, labl, (((1,), (0,)), ((), ())),
                preferred_element_type=jnp.float32)
            labs = jnp.sum(call * pick, axis=1, keepdims=True)
            uninit = jnp.where(labs.astype(jnp.int32) == kidx, 1.0, -1.0)
            obuf[j % 2, s] = jnp.where(has_inf, uninit, cos)
        out_copy(j).start()

    out_copy(nsteps - 2).wait()
    out_copy(nsteps - 1).wait()


def kernel(inputs, labels, class_avgs):
    b, t, d = inputs.shape
    k = class_avgs.shape[0]
    rows = (b * t) // _STREAMS
    nsteps = rows // _FTILE
    x3 = inputs.reshape(_STREAMS, rows, d)
    labs4 = labels.astype(jnp.int32).reshape(_STREAMS, nsteps, _LROWS, 128)

    out = pl.pallas_call(
        _pipeline_body,
        in_specs=[
            pl.BlockSpec(memory_space=pl.ANY),
            pl.BlockSpec(memory_space=pltpu.MemorySpace.VMEM),
            pl.BlockSpec(memory_space=pltpu.MemorySpace.VMEM),
        ],
        out_specs=pl.BlockSpec(memory_space=pl.ANY),
        out_shape=jax.ShapeDtypeStruct((_STREAMS, rows, k), jnp.float32),
        scratch_shapes=[
            pltpu.VMEM((_NBUF, _STREAMS, _FTILE, d), jnp.float32),
            pltpu.VMEM((2, _STREAMS, _FTILE, k), jnp.float32),
            pltpu.SemaphoreType.DMA((_NBUF, _STREAMS)),
            pltpu.SemaphoreType.DMA((2,)),
        ],
    )(x3, labs4, class_avgs)
    return out.reshape(b, t, k)
